# baseline (speedup 1.0000x reference)
import functools

import jax
import jax.numpy as jnp
from jax import lax
from jax.experimental import pallas as pl
from jax.experimental.pallas import tpu as pltpu
from jax.experimental.pallas import tpu_sc as plsc

B, S, TOPK, VOCAB = 16, 16, 4096, 50257
R = B * S
NC, NS, L = 2, 16, 16
NW = NC * NS
NCHUNKS = 2
BC = B // NCHUNKS
CR = BC * S
ROWS_PER_W = CR // NW
ROWPAD = 50304
NFILL = ROWPAD // L
NSCAT = TOPK // L


def _prep_body(v_ref, logv_ref, floor_ref):
    v = v_ref[...]
    logv_ref[...] = jnp.log(v + 1e-40)
    pmass = jnp.sum(v, axis=-1)
    rem = jnp.clip(1.0 - pmass, 1e-40, 1.0)
    fl = jnp.log(rem / (VOCAB - TOPK))
    floor_ref[...] = jnp.broadcast_to(fl[:, :, None], (B, S, L))


_prep = pl.pallas_call(
    _prep_body,
    out_shape=[
        jax.ShapeDtypeStruct((B, S, TOPK), jnp.float32),
        jax.ShapeDtypeStruct((B, S, L), jnp.float32),
    ],
)


@functools.partial(
    pl.kernel,
    out_type=jax.ShapeDtypeStruct((CR, ROWPAD), jnp.float32),
    mesh=plsc.VectorSubcoreMesh(core_axis_name="c", subcore_axis_name="s"),
    compiler_params=pltpu.CompilerParams(
        needs_layout_passes=False, use_tc_tiling_on_sc=False),
    scratch_types=[
        pltpu.VMEM((ROWPAD,), jnp.float32),
        pltpu.VMEM((ROWPAD,), jnp.float32),
        pltpu.VMEM((4, TOPK), jnp.int32),
        pltpu.VMEM((2, TOPK), jnp.float32),
        pltpu.VMEM((ROWS_PER_W, L), jnp.float32),
        pltpu.SemaphoreType.DMA,
        pltpu.SemaphoreType.DMA,
        pltpu.SemaphoreType.DMA((4,)),
        pltpu.SemaphoreType.DMA((2,)),
    ],
)
def _sc_scatter(logv_hbm, idx_hbm, floor_hbm, out_hbm,
                rb0, rb1, idxbuf, valbuf, floorbuf,
                sout0, sout1, sidx, sval):
    wid = lax.axis_index("s") * NC + lax.axis_index("c")
    r_flat0 = wid * ROWS_PER_W
    b = r_flat0 >> 4
    s0 = r_flat0 & 15
    rbs = (rb0, rb1)
    souts = (sout0, sout1)

    pltpu.sync_copy(floor_hbm.at[b, pl.ds(s0, ROWS_PER_W)], floorbuf)

    def full_fill(rb, splat):
        def fill(i, carry):
            rb[pl.ds(i * L, L)] = splat
            return carry
        lax.fori_loop(0, NFILL, fill, None, unroll=8)

    def scat_vals(rb, q, p):
        def scat(j, carry):
            iv = idxbuf[q, pl.ds(j * L, L)]
            vv = valbuf[p, pl.ds(j * L, L)]
            plsc.store_scatter(rb, [iv], vv)
            return carry
        lax.fori_loop(0, NSCAT, scat, None, unroll=8)

    def scat_reset(rb, q, splat):
        def scat(j, carry):
            iv = idxbuf[q, pl.ds(j * L, L)]
            plsc.store_scatter(rb, [iv], splat)
            return carry
        lax.fori_loop(0, NSCAT, scat, None, unroll=8)

    cp_idx = [None] * 4
    cp_val = [None] * 2
    cp_out = [None] * 2
    for r in range(2):
        cp_idx[r] = pltpu.async_copy(
            idx_hbm.at[b, s0 + r], idxbuf.at[r], sidx.at[r])
        cp_val[r] = pltpu.async_copy(
            logv_hbm.at[b, s0 + r], valbuf.at[r], sval.at[r])

    for r in range(ROWS_PER_W):
        p = r % 2
        q = r % 4
        rb = rbs[p]
        splat = floorbuf[r]
        if r >= 2:
            cp_out[p].wait()
            prev = floorbuf[r - 2]
            same = jnp.max(jnp.abs(splat - prev)) == 0.0
            lax.cond(same,
                     lambda: scat_reset(rb, (r - 2) % 4, splat),
                     lambda: full_fill(rb, splat))
        else:
            full_fill(rb, splat)
        if r + 1 < ROWS_PER_W:
            nq = (r + 1) % 4
            cp_idx[nq] = pltpu.async_copy(
                idx_hbm.at[b, s0 + r + 1], idxbuf.at[nq], sidx.at[nq])
            cp_val[1 - p] = pltpu.async_copy(
                logv_hbm.at[b, s0 + r + 1], valbuf.at[1 - p], sval.at[1 - p])
        cp_idx[q].wait()
        cp_val[p].wait()
        scat_vals(rb, q, p)
        cp_out[p] = pltpu.async_copy(rb, out_hbm.at[r_flat0 + r], souts[p])
    cp_out[0].wait()
    cp_out[1].wait()


def kernel(topk_values, topk_indices):
    logv, floor = _prep(topk_values)
    final = jnp.zeros((B, S, VOCAB), jnp.float32)
    for c in range(NCHUNKS):
        sl = slice(c * BC, (c + 1) * BC)
        o = _sc_scatter(logv[sl], topk_indices[sl], floor[sl])
        part = o.reshape(BC, S, ROWPAD)[:, :, :VOCAB]
        final = lax.dynamic_update_slice(final, part, (c * BC, 0, 0))
    return final


# speedup vs baseline: 1.4536x; 1.4536x over previous
import functools

import jax
import jax.numpy as jnp
from jax import lax
from jax.experimental import pallas as pl
from jax.experimental.pallas import tpu as pltpu
from jax.experimental.pallas import tpu_sc as plsc

B, S, TOPK, VOCAB = 16, 16, 4096, 50257
R = B * S
NC, NS, L = 2, 16, 16
NW = NC * NS
ROWS_PER_W = R // NW
ROWPAD = 50304
NFILL = ROWPAD // L
NSCAT = TOPK // L


def _prep_body(v_ref, i_ref, logv_ref, idx_ref, floor_ref):
    v = v_ref[...]
    logv_ref[...] = jnp.log(v + 1e-40).reshape(R * 4, 8, 128)
    idx_ref[...] = i_ref[...].reshape(R * 4, 8, 128)
    pmass = jnp.sum(v, axis=-1)
    rem = jnp.clip(1.0 - pmass, 1e-40, 1.0)
    fl = jnp.log(rem / (VOCAB - TOPK))
    floor_ref[...] = jnp.broadcast_to(
        fl.reshape(R)[:, None], (R, 128)).reshape(NW, 8, 128)


_prep = pl.pallas_call(
    _prep_body,
    out_shape=[
        jax.ShapeDtypeStruct((R * 4, 8, 128), jnp.float32),
        jax.ShapeDtypeStruct((R * 4, 8, 128), jnp.int32),
        jax.ShapeDtypeStruct((NW, 8, 128), jnp.float32),
    ],
)


@functools.partial(
    pl.kernel,
    out_type=jax.ShapeDtypeStruct((R, ROWPAD), jnp.float32),
    mesh=plsc.VectorSubcoreMesh(core_axis_name="c", subcore_axis_name="s"),
    compiler_params=pltpu.CompilerParams(
        needs_layout_passes=False, use_tc_tiling_on_sc=False),
    scratch_types=[
        pltpu.VMEM((ROWPAD,), jnp.float32),
        pltpu.VMEM((ROWPAD,), jnp.float32),
        pltpu.VMEM((4, 4, 8, 128), jnp.int32),
        pltpu.VMEM((2, 4, 8, 128), jnp.float32),
        pltpu.VMEM((8, 128), jnp.float32),
        pltpu.SemaphoreType.DMA,
        pltpu.SemaphoreType.DMA,
        pltpu.SemaphoreType.DMA((4,)),
        pltpu.SemaphoreType.DMA((2,)),
    ],
)
def _sc_scatter(logv_hbm, idx_hbm, floor_hbm, out_hbm,
                rb0, rb1, idxbuf, valbuf, floorbuf,
                sout0, sout1, sidx, sval):
    wid = lax.axis_index("s") * NC + lax.axis_index("c")
    r_flat0 = wid * ROWS_PER_W
    rbs = (rb0, rb1)
    souts = (sout0, sout1)

    pltpu.sync_copy(floor_hbm.at[wid], floorbuf)

    def full_fill(rb, splat):
        def fill(i, carry):
            rb[pl.ds(i * L, L)] = splat
            return carry
        lax.fori_loop(0, NFILL, fill, None, unroll=8)

    def scat_vals(rb, q, p):
        def scat(j, carry):
            c = j >> 6
            m = (j >> 3) & 7
            l0 = (j & 7) * L
            iv = idxbuf[q, c, m, pl.ds(l0, L)]
            vv = valbuf[p, c, m, pl.ds(l0, L)]
            plsc.store_scatter(rb, [iv], vv)
            return carry
        lax.fori_loop(0, NSCAT, scat, None, unroll=8)

    def scat_reset(rb, q, splat):
        def scat(j, carry):
            c = j >> 6
            m = (j >> 3) & 7
            l0 = (j & 7) * L
            iv = idxbuf[q, c, m, pl.ds(l0, L)]
            plsc.store_scatter(rb, [iv], splat)
            return carry
        lax.fori_loop(0, NSCAT, scat, None, unroll=8)

    cp_idx = [None] * 4
    cp_val = [None] * 2
    cp_out = [None] * 2
    for r in range(2):
        cp_idx[r] = pltpu.async_copy(
            idx_hbm.at[pl.ds((r_flat0 + r) * 4, 4)], idxbuf.at[r],
            sidx.at[r])
        cp_val[r] = pltpu.async_copy(
            logv_hbm.at[pl.ds((r_flat0 + r) * 4, 4)], valbuf.at[r],
            sval.at[r])

    for r in range(ROWS_PER_W):
        p = r % 2
        q = r % 4
        rb = rbs[p]
        splat = floorbuf[r, pl.ds(0, L)]
        if r >= 2:
            cp_out[p].wait()
            prev = floorbuf[r - 2, pl.ds(0, L)]
            same = jnp.max(jnp.abs(splat - prev)) == 0.0
            lax.cond(same,
                     lambda: scat_reset(rb, (r - 2) % 4, splat),
                     lambda: full_fill(rb, splat))
        else:
            full_fill(rb, splat)
        if r + 1 < ROWS_PER_W:
            nq = (r + 1) % 4
            cp_idx[nq] = pltpu.async_copy(
                idx_hbm.at[pl.ds((r_flat0 + r + 1) * 4, 4)], idxbuf.at[nq],
                sidx.at[nq])
            cp_val[1 - p] = pltpu.async_copy(
                logv_hbm.at[pl.ds((r_flat0 + r + 1) * 4, 4)],
                valbuf.at[1 - p], sval.at[1 - p])
        cp_idx[q].wait()
        cp_val[p].wait()
        scat_vals(rb, q, p)
        cp_out[p] = pltpu.async_copy(rb, out_hbm.at[r_flat0 + r], souts[p])
    cp_out[0].wait()
    cp_out[1].wait()


def kernel(topk_values, topk_indices):
    logv, idx, floor = _prep(topk_values, topk_indices)
    out = _sc_scatter(logv, idx, floor)
    return out.reshape(B, S, ROWPAD)[:, :, :VOCAB]


# speedup vs baseline: 1.4613x; 1.0053x over previous
import functools

import jax
import jax.numpy as jnp
from jax import lax
from jax.experimental import pallas as pl
from jax.experimental.pallas import tpu as pltpu
from jax.experimental.pallas import tpu_sc as plsc

B, S, TOPK, VOCAB = 16, 16, 4096, 50257
R = B * S
NC, NS, L = 2, 16, 16
NW = NC * NS
ROWS_PER_W = R // NW
ROWPAD = 50304
NFILL = ROWPAD // L
NSCAT = TOPK // L


def _prep_body(v_ref, i_ref, logv_ref, idx_ref, floor_ref):
    v = v_ref[...]
    rblk = _PREP_BBLK * S
    logv_ref[...] = jnp.log(v + 1e-40).reshape(rblk * 4, 8, 128)
    idx_ref[...] = i_ref[...].reshape(rblk * 4, 8, 128)
    pmass = jnp.sum(v, axis=-1)
    rem = jnp.clip(1.0 - pmass, 1e-40, 1.0)
    fl = jnp.log(rem / (VOCAB - TOPK))
    floor_ref[...] = jnp.broadcast_to(
        fl.reshape(rblk)[:, None], (rblk, 128)).reshape(rblk // 8, 8, 128)


_PREP_BBLK = 4
_prep = pl.pallas_call(
    _prep_body,
    grid=(B // _PREP_BBLK,),
    in_specs=[
        pl.BlockSpec((_PREP_BBLK, S, TOPK), lambda i: (i, 0, 0)),
        pl.BlockSpec((_PREP_BBLK, S, TOPK), lambda i: (i, 0, 0)),
    ],
    out_specs=[
        pl.BlockSpec((_PREP_BBLK * S * 4, 8, 128), lambda i: (i, 0, 0)),
        pl.BlockSpec((_PREP_BBLK * S * 4, 8, 128), lambda i: (i, 0, 0)),
        pl.BlockSpec((_PREP_BBLK * S // 8, 8, 128), lambda i: (i, 0, 0)),
    ],
    out_shape=[
        jax.ShapeDtypeStruct((R * 4, 8, 128), jnp.float32),
        jax.ShapeDtypeStruct((R * 4, 8, 128), jnp.int32),
        jax.ShapeDtypeStruct((NW, 8, 128), jnp.float32),
    ],
)


@functools.partial(
    pl.kernel,
    out_type=jax.ShapeDtypeStruct((R, ROWPAD), jnp.float32),
    mesh=plsc.VectorSubcoreMesh(core_axis_name="c", subcore_axis_name="s"),
    compiler_params=pltpu.CompilerParams(
        needs_layout_passes=False, use_tc_tiling_on_sc=False),
    scratch_types=[
        pltpu.VMEM((ROWPAD,), jnp.float32),
        pltpu.VMEM((ROWPAD,), jnp.float32),
        pltpu.VMEM((4, 4, 8, 128), jnp.int32),
        pltpu.VMEM((2, 4, 8, 128), jnp.float32),
        pltpu.VMEM((8, 128), jnp.float32),
        pltpu.SemaphoreType.DMA,
        pltpu.SemaphoreType.DMA,
        pltpu.SemaphoreType.DMA((4,)),
        pltpu.SemaphoreType.DMA((2,)),
    ],
)
def _sc_scatter(logv_hbm, idx_hbm, floor_hbm, out_hbm,
                rb0, rb1, idxbuf, valbuf, floorbuf,
                sout0, sout1, sidx, sval):
    wid = lax.axis_index("s") * NC + lax.axis_index("c")
    r_flat0 = wid * ROWS_PER_W
    rbs = (rb0, rb1)
    souts = (sout0, sout1)

    pltpu.sync_copy(floor_hbm.at[wid], floorbuf)

    def full_fill(rb, splat):
        def fill(i, carry):
            rb[pl.ds(i * L, L)] = splat
            return carry
        lax.fori_loop(0, NFILL, fill, None, unroll=8)

    def scat_vals(rb, q, p):
        def scat(j, carry):
            c = j >> 6
            m = (j >> 3) & 7
            l0 = (j & 7) * L
            iv = idxbuf[q, c, m, pl.ds(l0, L)]
            vv = valbuf[p, c, m, pl.ds(l0, L)]
            plsc.store_scatter(rb, [iv], vv)
            return carry
        lax.fori_loop(0, NSCAT, scat, None, unroll=8)

    def scat_reset(rb, q, splat):
        def scat(j, carry):
            c = j >> 6
            m = (j >> 3) & 7
            l0 = (j & 7) * L
            iv = idxbuf[q, c, m, pl.ds(l0, L)]
            plsc.store_scatter(rb, [iv], splat)
            return carry
        lax.fori_loop(0, NSCAT, scat, None, unroll=8)

    cp_idx = [None] * 4
    cp_val = [None] * 2
    cp_out = [None] * 2
    for r in range(2):
        cp_idx[r] = pltpu.async_copy(
            idx_hbm.at[pl.ds((r_flat0 + r) * 4, 4)], idxbuf.at[r],
            sidx.at[r])
        cp_val[r] = pltpu.async_copy(
            logv_hbm.at[pl.ds((r_flat0 + r) * 4, 4)], valbuf.at[r],
            sval.at[r])

    for r in range(ROWS_PER_W):
        p = r % 2
        q = r % 4
        rb = rbs[p]
        splat = floorbuf[r, pl.ds(0, L)]
        if r >= 2:
            cp_out[p].wait()
            prev = floorbuf[r - 2, pl.ds(0, L)]
            same = jnp.max(jnp.abs(splat - prev)) == 0.0
            lax.cond(same,
                     lambda: scat_reset(rb, (r - 2) % 4, splat),
                     lambda: full_fill(rb, splat))
        else:
            full_fill(rb, splat)
        if r + 1 < ROWS_PER_W:
            nq = (r + 1) % 4
            cp_idx[nq] = pltpu.async_copy(
                idx_hbm.at[pl.ds((r_flat0 + r + 1) * 4, 4)], idxbuf.at[nq],
                sidx.at[nq])
            cp_val[1 - p] = pltpu.async_copy(
                logv_hbm.at[pl.ds((r_flat0 + r + 1) * 4, 4)],
                valbuf.at[1 - p], sval.at[1 - p])
        cp_idx[q].wait()
        cp_val[p].wait()
        scat_vals(rb, q, p)
        cp_out[p] = pltpu.async_copy(rb, out_hbm.at[r_flat0 + r], souts[p])
    cp_out[0].wait()
    cp_out[1].wait()


def kernel(topk_values, topk_indices):
    logv, idx, floor = _prep(topk_values, topk_indices)
    out = _sc_scatter(logv, idx, floor)
    return out.reshape(B, S, ROWPAD)[:, :, :VOCAB]


# speedup vs baseline: 2.5801x; 1.7656x over previous
import functools

import jax
import jax.numpy as jnp
from jax import lax
from jax.experimental import pallas as pl
from jax.experimental.pallas import tpu as pltpu
from jax.experimental.pallas import tpu_sc as plsc

B, S, TOPK, VOCAB = 16, 16, 4096, 50257
R = B * S
NC, NS, L = 2, 16, 16
NW = NC * NS
ROWS_PER_W = R // NW
ROWPAD = 50304
NFILL = ROWPAD // L
RBUF = 51200
VFULL = 50176
NSCAT = TOPK // L


def _prep_body(v_ref, i_ref, logv_ref, idx_ref, floor_ref):
    v = v_ref[...]
    logv_ref[...] = jnp.log(v + 1e-40).reshape(R * 4, 8, 128)
    idx_ref[...] = i_ref[...].reshape(R * 4, 8, 128)
    pmass = jnp.sum(v, axis=-1)
    rem = jnp.clip(1.0 - pmass, 1e-40, 1.0)
    fl = jnp.log(rem / (VOCAB - TOPK))
    floor_ref[...] = jnp.broadcast_to(
        fl.reshape(R)[:, None], (R, 128)).reshape(NW, 8, 128)


_prep = pl.pallas_call(
    _prep_body,
    out_shape=[
        jax.ShapeDtypeStruct((R * 4, 8, 128), jnp.float32),
        jax.ShapeDtypeStruct((R * 4, 8, 128), jnp.int32),
        jax.ShapeDtypeStruct((NW, 8, 128), jnp.float32),
    ],
)


@functools.partial(
    pl.kernel,
    out_type=[
        jax.ShapeDtypeStruct((B, S, VOCAB), jnp.float32),
        jax.ShapeDtypeStruct((NW, 8, 128), jnp.float32),
    ],
    mesh=plsc.VectorSubcoreMesh(core_axis_name="c", subcore_axis_name="s"),
    compiler_params=pltpu.CompilerParams(
        needs_layout_passes=False, use_tc_tiling_on_sc=True),
    scratch_types=[
        pltpu.VMEM((RBUF,), jnp.float32),
        pltpu.VMEM((RBUF,), jnp.float32),
        pltpu.VMEM((4, 4, 8, 128), jnp.int32),
        pltpu.VMEM((2, 4, 8, 128), jnp.float32),
        pltpu.VMEM((8, 128), jnp.float32),
        pltpu.VMEM((8, 128), jnp.float32),
        pltpu.SemaphoreType.DMA,
        pltpu.SemaphoreType.DMA,
        pltpu.SemaphoreType.DMA((4,)),
        pltpu.SemaphoreType.DMA((2,)),
    ],
)
def _sc_scatter(logv_hbm, idx_hbm, floor_hbm, out_hbm, tail_hbm,
                rb0, rb1, idxbuf, valbuf, floorbuf, tailtile,
                sout0, sout1, sidx, sval):
    wid = lax.axis_index("s") * NC + lax.axis_index("c")
    r_flat0 = wid * ROWS_PER_W
    rbs = (rb0, rb1)
    souts = (sout0, sout1)

    pltpu.sync_copy(floor_hbm.at[wid], floorbuf)

    def full_fill(rb, splat):
        def fill(i, carry):
            rb[pl.ds(i * L, L)] = splat
            return carry
        lax.fori_loop(0, NFILL, fill, None, unroll=8)

    def scat_vals(rb, q, p):
        def scat(j, carry):
            c = j >> 6
            m = (j >> 3) & 7
            l0 = (j & 7) * L
            iv = idxbuf[q, c, m, pl.ds(l0, L)]
            vv = valbuf[p, c, m, pl.ds(l0, L)]
            plsc.store_scatter(rb, [iv], vv)
            return carry
        lax.fori_loop(0, NSCAT, scat, None, unroll=8)

    def scat_reset(rb, q, splat):
        def scat(j, carry):
            c = j >> 6
            m = (j >> 3) & 7
            l0 = (j & 7) * L
            iv = idxbuf[q, c, m, pl.ds(l0, L)]
            plsc.store_scatter(rb, [iv], splat)
            return carry
        lax.fori_loop(0, NSCAT, scat, None, unroll=8)

    cp_idx = [None] * 4
    cp_val = [None] * 2
    cp_out = [None] * 2
    for r in range(2):
        cp_idx[r] = pltpu.async_copy(
            idx_hbm.at[pl.ds((r_flat0 + r) * 4, 4)], idxbuf.at[r],
            sidx.at[r])
        cp_val[r] = pltpu.async_copy(
            logv_hbm.at[pl.ds((r_flat0 + r) * 4, 4)], valbuf.at[r],
            sval.at[r])

    for r in range(ROWS_PER_W):
        p = r % 2
        q = r % 4
        rb = rbs[p]
        r_flat = r_flat0 + r
        bb = r_flat >> 4
        ss = r_flat & 15
        splat = floorbuf[r, pl.ds(0, L)]
        if r >= 2:
            cp_out[p].wait()
            prev = floorbuf[r - 2, pl.ds(0, L)]
            same = jnp.max(jnp.abs(splat - prev)) == 0.0
            lax.cond(same,
                     lambda: scat_reset(rb, (r - 2) % 4, splat),
                     lambda: full_fill(rb, splat))
        else:
            full_fill(rb, splat)
        if r + 1 < ROWS_PER_W:
            nq = (r + 1) % 4
            cp_idx[nq] = pltpu.async_copy(
                idx_hbm.at[pl.ds((r_flat0 + r + 1) * 4, 4)], idxbuf.at[nq],
                sidx.at[nq])
            cp_val[1 - p] = pltpu.async_copy(
                logv_hbm.at[pl.ds((r_flat0 + r + 1) * 4, 4)],
                valbuf.at[1 - p], sval.at[1 - p])
        cp_idx[q].wait()
        cp_val[p].wait()
        scat_vals(rb, q, p)
        cp_out[p] = pltpu.async_copy(
            rb.at[pl.ds(0, VFULL)], out_hbm.at[bb, ss, pl.ds(0, VFULL)],
            souts[p])
        for i in range(8):
            tailtile[r, pl.ds(i * L, L)] = rb[pl.ds(VFULL + i * L, L)]
    pltpu.sync_copy(tailtile, tail_hbm.at[wid])
    cp_out[0].wait()
    cp_out[1].wait()


def _merge_body(m_ref, t_ref, o_ref):
    o_ref[...] = t_ref[...]


_merge = pl.pallas_call(
    _merge_body,
    grid=(1,),
    in_specs=[
        pl.BlockSpec((B, S, 128), lambda i: (0, 0, VFULL // 128)),
        pl.BlockSpec((B, S, 128), lambda i: (0, 0, 0)),
    ],
    out_specs=pl.BlockSpec((B, S, 128), lambda i: (0, 0, VFULL // 128)),
    out_shape=jax.ShapeDtypeStruct((B, S, VOCAB), jnp.float32),
    input_output_aliases={0: 0},
)


def kernel(topk_values, topk_indices):
    logv, idx, floor = _prep(topk_values, topk_indices)
    main, tail = _sc_scatter(logv, idx, floor)
    return _merge(main, tail.reshape(B, S, 128))
